# Initial kernel scaffold; baseline (speedup 1.0000x reference)
import jax
import jax.numpy as jnp
from jax.experimental import pallas as pl


def kernel(x, adj, W1, W2):
    raise NotImplementedError("write your pallas kernel here")


# speedup vs baseline: 26.4056x; 26.4056x over previous
import functools

import jax
import jax.numpy as jnp
from jax import lax
from jax.experimental import pallas as pl
from jax.experimental.pallas import tpu as pltpu
from jax.experimental.pallas import tpu_sc as plsc

N_NODES = 10000
N_EDGES = 320000
D_MSG = 16

NP = 10240
EP = 327680
GSZ = 128
GROUPS = EP // GSZ
NC, NS = 2, 16
NW = NC * NS
GPW = GROUPS // NW
CH = 8
NCHUNK = GPW // CH
RPW = NP // NS

_mesh = plsc.VectorSubcoreMesh(core_axis_name="c", subcore_axis_name="s")


def _msg_body(hs_hbm, src_hbm, dst_hbm, zeros_hbm, out_hbm,
              srcv, dstv, rowsbuf, acc_sh, gsem, ssem):
    cid = lax.axis_index("c")
    sid = lax.axis_index("s")
    wid = sid * NC + cid

    pltpu.sync_copy(zeros_hbm.at[pl.ds(sid * RPW, RPW)],
                    acc_sh.at[pl.ds(sid * RPW, RPW)])
    plsc.subcore_barrier()

    @pl.loop(0, NCHUNK)
    def _chunk(c):
        goff = wid * GPW + c * CH
        pltpu.sync_copy(src_hbm.at[pl.ds(goff, CH)], srcv)
        pltpu.sync_copy(dst_hbm.at[pl.ds(goff, CH)], dstv)
        gd = [pltpu.async_copy(hs_hbm.at[srcv.at[j]], rowsbuf.at[j], gsem)
              for j in range(CH)]
        for d_ in gd:
            d_.wait()
        sd = [pltpu.async_copy(rowsbuf.at[j], acc_sh.at[dstv.at[j]], ssem,
                               add=True)
              for j in range(CH)]
        for d_ in sd:
            d_.wait()

    plsc.subcore_barrier()
    pltpu.sync_copy(acc_sh.at[pl.ds(sid * RPW, RPW)],
                    out_hbm.at[cid].at[pl.ds(sid * RPW, RPW)])


_msg_kernel = pl.kernel(
    _msg_body,
    out_type=jax.ShapeDtypeStruct((NC, NP, D_MSG), jnp.float32),
    mesh=_mesh,
    scratch_types=[
        pltpu.VMEM((CH, GSZ), jnp.int32),
        pltpu.VMEM((CH, GSZ), jnp.int32),
        pltpu.VMEM((CH, GSZ, D_MSG), jnp.float32),
        pltpu.VMEM_SHARED((NP, D_MSG), jnp.float32),
        pltpu.SemaphoreType.DMA,
        pltpu.SemaphoreType.DMA,
    ],
    compiler_params=pltpu.CompilerParams(use_tc_tiling_on_sc=False),
)


def _mm1_body(x_ref, w_ref, o_ref):
    o_ref[...] = jnp.dot(x_ref[...], w_ref[...],
                         preferred_element_type=jnp.float32)


def _prep_body(degp_ref, h1_ref, dinv_ref, hs_ref):
    deg = degp_ref[0] + degp_ref[1] + 1.0
    dinv = lax.rsqrt(jnp.maximum(deg, 1.0))
    dinv_ref[...] = dinv
    hs_ref[...] = h1_ref[...] * dinv


def _mid_body(acc_ref, hs_ref, dinv_ref, g_ref):
    t = (acc_ref[0] + acc_ref[1] + hs_ref[...]) * dinv_ref[...]
    g_ref[...] = jnp.maximum(t, 0.0) * dinv_ref[...]


def _fin_body(acc_ref, g_ref, dinv_ref, w2_ref, o_ref):
    a = (acc_ref[0] + acc_ref[1] + g_ref[...]) * dinv_ref[...]
    o_ref[...] = jnp.dot(a, w2_ref[...], preferred_element_type=jnp.float32)


def kernel(x, adj, W1, W2):
    n = x.shape[0]
    e = adj.shape[1]
    d_out = W2.shape[1]
    src = adj[0].astype(jnp.int32)
    dst = adj[1].astype(jnp.int32)
    pad = EP - e
    src2d = jnp.concatenate([src, jnp.full((pad,), n, jnp.int32)]).reshape(
        GROUPS, GSZ)
    dst2d = jnp.concatenate([dst, jnp.full((pad,), n, jnp.int32)]).reshape(
        GROUPS, GSZ)
    zeros = jnp.zeros((NP, D_MSG), jnp.float32)
    ones = jnp.ones((NP, D_MSG), jnp.float32)

    degp = _msg_kernel(ones, src2d, dst2d, zeros)
    h1 = pl.pallas_call(
        _mm1_body,
        out_shape=jax.ShapeDtypeStruct((n, D_MSG), jnp.float32),
    )(x, W1)
    h1p = jnp.concatenate([h1, jnp.zeros((NP - n, D_MSG), jnp.float32)])

    dinvb, hs1 = pl.pallas_call(
        _prep_body,
        out_shape=(jax.ShapeDtypeStruct((NP, D_MSG), jnp.float32),
                   jax.ShapeDtypeStruct((NP, D_MSG), jnp.float32)),
    )(degp, h1p)

    acc1 = _msg_kernel(hs1, src2d, dst2d, zeros)

    g = pl.pallas_call(
        _mid_body,
        out_shape=jax.ShapeDtypeStruct((NP, D_MSG), jnp.float32),
    )(acc1, hs1, dinvb)

    acc2 = _msg_kernel(g, src2d, dst2d, zeros)

    out = pl.pallas_call(
        _fin_body,
        out_shape=jax.ShapeDtypeStruct((NP, d_out), jnp.float32),
    )(acc2, g, dinvb, W2)
    return out[:n]


# speedup vs baseline: 37.4313x; 1.4175x over previous
import functools

import jax
import jax.numpy as jnp
from jax import lax
from jax.experimental import pallas as pl
from jax.experimental.pallas import tpu as pltpu
from jax.experimental.pallas import tpu_sc as plsc

N_NODES = 10000
N_EDGES = 320000
D_MSG = 16

NP = 10240
EP = 327680
GSZ = 128
GROUPS = EP // GSZ
NC, NS = 2, 16
NW = NC * NS
GPW = GROUPS // NW
CH = 8
NCHUNK = GPW // CH
RPW = NP // NS

_mesh = plsc.VectorSubcoreMesh(core_axis_name="c", subcore_axis_name="s")


def _msg_body(hs_hbm, src_hbm, dst_hbm, zeros_hbm, out_hbm,
              srcv, dstv, rowsbuf, acc_sh, gsem, ssem):
    cid = lax.axis_index("c")
    sid = lax.axis_index("s")
    wid = sid * NC + cid
    base = wid * GPW

    pltpu.sync_copy(zeros_hbm.at[pl.ds(sid * RPW, RPW)],
                    acc_sh.at[pl.ds(sid * RPW, RPW)])
    plsc.subcore_barrier()

    def load_idx(c, b):
        pltpu.sync_copy(src_hbm.at[pl.ds(base + c * CH, CH)], srcv.at[b])
        pltpu.sync_copy(dst_hbm.at[pl.ds(base + c * CH, CH)], dstv.at[b])

    def fire_gathers(b):
        return [pltpu.async_copy(hs_hbm.at[srcv.at[b].at[j]],
                                 rowsbuf.at[b].at[j], gsem)
                for j in range(CH)]

    def fire_scatters(b):
        return [pltpu.async_copy(rowsbuf.at[b].at[j],
                                 acc_sh.at[dstv.at[b].at[j]], ssem, add=True)
                for j in range(CH)]

    load_idx(0, 0)
    g = fire_gathers(0)
    s_prev = None
    for c in range(NCHUNK):
        b = c % 2
        for d_ in g:
            d_.wait()
        s = fire_scatters(b)
        if s_prev is not None:
            for d_ in s_prev:
                d_.wait()
        if c + 1 < NCHUNK:
            load_idx(c + 1, 1 - b)
            g = fire_gathers(1 - b)
        s_prev = s
    for d_ in s_prev:
        d_.wait()

    plsc.subcore_barrier()
    pltpu.sync_copy(acc_sh.at[pl.ds(sid * RPW, RPW)],
                    out_hbm.at[cid].at[pl.ds(sid * RPW, RPW)])


_msg_kernel = pl.kernel(
    _msg_body,
    out_type=jax.ShapeDtypeStruct((NC, NP, D_MSG), jnp.float32),
    mesh=_mesh,
    scratch_types=[
        pltpu.VMEM((2, CH, GSZ), jnp.int32),
        pltpu.VMEM((2, CH, GSZ), jnp.int32),
        pltpu.VMEM((2, CH, GSZ, D_MSG), jnp.float32),
        pltpu.VMEM_SHARED((NP, D_MSG), jnp.float32),
        pltpu.SemaphoreType.DMA,
        pltpu.SemaphoreType.DMA,
    ],
    compiler_params=pltpu.CompilerParams(use_tc_tiling_on_sc=False),
)


def _deg_body(dst_hbm, ones_hbm, zeros_hbm, out_hbm, dstv, onesb, acc_sh,
              ssem):
    cid = lax.axis_index("c")
    sid = lax.axis_index("s")
    wid = sid * NC + cid
    base = wid * GPW

    pltpu.sync_copy(zeros_hbm.at[pl.ds(sid * RPW, RPW)],
                    acc_sh.at[pl.ds(sid * RPW, RPW)])
    pltpu.sync_copy(ones_hbm, onesb)
    plsc.subcore_barrier()

    s_prev = None
    for c in range(NCHUNK):
        b = c % 2
        pltpu.sync_copy(dst_hbm.at[pl.ds(base + c * CH, CH)], dstv.at[b])
        s = [pltpu.async_copy(onesb, acc_sh.at[dstv.at[b].at[j]], ssem,
                              add=True)
             for j in range(CH)]
        if s_prev is not None:
            for d_ in s_prev:
                d_.wait()
        s_prev = s
    for d_ in s_prev:
        d_.wait()

    plsc.subcore_barrier()
    pltpu.sync_copy(acc_sh.at[pl.ds(sid * RPW, RPW)],
                    out_hbm.at[cid].at[pl.ds(sid * RPW, RPW)])


_deg_kernel = pl.kernel(
    _deg_body,
    out_type=jax.ShapeDtypeStruct((NC, NP, D_MSG), jnp.float32),
    mesh=_mesh,
    scratch_types=[
        pltpu.VMEM((2, CH, GSZ), jnp.int32),
        pltpu.VMEM((GSZ, D_MSG), jnp.float32),
        pltpu.VMEM_SHARED((NP, D_MSG), jnp.float32),
        pltpu.SemaphoreType.DMA,
    ],
    compiler_params=pltpu.CompilerParams(use_tc_tiling_on_sc=False),
)


def _mm1_body(x_ref, w_ref, o_ref):
    o_ref[...] = jnp.dot(x_ref[...], w_ref[...],
                         preferred_element_type=jnp.float32)


def _prep_body(degp_ref, h1_ref, dinv_ref, hs_ref):
    deg = degp_ref[0] + degp_ref[1] + 1.0
    dinv = lax.rsqrt(jnp.maximum(deg, 1.0))
    dinv_ref[...] = dinv
    hs_ref[...] = h1_ref[...] * dinv


def _mid_body(acc_ref, hs_ref, dinv_ref, g_ref):
    t = (acc_ref[0] + acc_ref[1] + hs_ref[...]) * dinv_ref[...]
    g_ref[...] = jnp.maximum(t, 0.0) * dinv_ref[...]


def _fin_body(acc_ref, g_ref, dinv_ref, w2_ref, o_ref):
    a = (acc_ref[0] + acc_ref[1] + g_ref[...]) * dinv_ref[...]
    o_ref[...] = jnp.dot(a, w2_ref[...], preferred_element_type=jnp.float32)


def kernel(x, adj, W1, W2):
    n = x.shape[0]
    e = adj.shape[1]
    d_out = W2.shape[1]
    src = adj[0].astype(jnp.int32)
    dst = adj[1].astype(jnp.int32)
    pad = EP - e
    src2d = jnp.concatenate([src, jnp.full((pad,), n, jnp.int32)]).reshape(
        GROUPS, GSZ)
    dst2d = jnp.concatenate([dst, jnp.full((pad,), n, jnp.int32)]).reshape(
        GROUPS, GSZ)
    zeros = jnp.zeros((NP, D_MSG), jnp.float32)
    ones = jnp.ones((GSZ, D_MSG), jnp.float32)

    degp = _deg_kernel(dst2d, ones, zeros)
    h1 = pl.pallas_call(
        _mm1_body,
        out_shape=jax.ShapeDtypeStruct((n, D_MSG), jnp.float32),
    )(x, W1)
    h1p = jnp.concatenate([h1, jnp.zeros((NP - n, D_MSG), jnp.float32)])

    dinvb, hs1 = pl.pallas_call(
        _prep_body,
        out_shape=(jax.ShapeDtypeStruct((NP, D_MSG), jnp.float32),
                   jax.ShapeDtypeStruct((NP, D_MSG), jnp.float32)),
    )(degp, h1p)

    acc1 = _msg_kernel(hs1, src2d, dst2d, zeros)

    g = pl.pallas_call(
        _mid_body,
        out_shape=jax.ShapeDtypeStruct((NP, D_MSG), jnp.float32),
    )(acc1, hs1, dinvb)

    acc2 = _msg_kernel(g, src2d, dst2d, zeros)

    out = pl.pallas_call(
        _fin_body,
        out_shape=jax.ShapeDtypeStruct((NP, d_out), jnp.float32),
    )(acc2, g, dinvb, W2)
    return out[:n]


# speedup vs baseline: 52.3921x; 1.3997x over previous
import functools

import jax
import jax.numpy as jnp
from jax import lax
from jax.experimental import pallas as pl
from jax.experimental.pallas import tpu as pltpu
from jax.experimental.pallas import tpu_sc as plsc

N_NODES = 10000
N_EDGES = 320000
D_MSG = 16

NP = 10240
EP = 327680
GSZ = 128
GROUPS = EP // GSZ
NC, NS = 2, 16
NW = NC * NS
GPW = GROUPS // NW
CH = 8
NCHUNK = GPW // CH
RPW = NP // NS

_mesh = plsc.VectorSubcoreMesh(core_axis_name="c", subcore_axis_name="s")


def _msg_body(hs_hbm, src_hbm, dst_hbm, zeros_hbm, out_hbm,
              srcv, dstv, rowsbuf, acc_sh, table_sh, gsem, ssem):
    cid = lax.axis_index("c")
    sid = lax.axis_index("s")
    wid = sid * NC + cid
    base = wid * GPW

    pltpu.sync_copy(zeros_hbm.at[pl.ds(sid * RPW, RPW)],
                    acc_sh.at[pl.ds(sid * RPW, RPW)])
    pltpu.sync_copy(hs_hbm.at[pl.ds(sid * RPW, RPW)],
                    table_sh.at[pl.ds(sid * RPW, RPW)])
    plsc.subcore_barrier()

    def load_idx(c, b):
        pltpu.sync_copy(src_hbm.at[pl.ds(base + c * CH, CH)], srcv.at[b])
        pltpu.sync_copy(dst_hbm.at[pl.ds(base + c * CH, CH)], dstv.at[b])

    def fire_gathers(b):
        return [pltpu.async_copy(table_sh.at[srcv.at[b].at[j]],
                                 rowsbuf.at[b].at[j], gsem)
                for j in range(CH)]

    def fire_scatters(b):
        return [pltpu.async_copy(rowsbuf.at[b].at[j],
                                 acc_sh.at[dstv.at[b].at[j]], ssem, add=True)
                for j in range(CH)]

    load_idx(0, 0)
    g = fire_gathers(0)
    s_prev = None
    for c in range(NCHUNK):
        b = c % 2
        for d_ in g:
            d_.wait()
        s = fire_scatters(b)
        if s_prev is not None:
            for d_ in s_prev:
                d_.wait()
        if c + 1 < NCHUNK:
            load_idx(c + 1, 1 - b)
            g = fire_gathers(1 - b)
        s_prev = s
    for d_ in s_prev:
        d_.wait()

    plsc.subcore_barrier()
    pltpu.sync_copy(acc_sh.at[pl.ds(sid * RPW, RPW)],
                    out_hbm.at[cid].at[pl.ds(sid * RPW, RPW)])


_msg_kernel = pl.kernel(
    _msg_body,
    out_type=jax.ShapeDtypeStruct((NC, NP, D_MSG), jnp.float32),
    mesh=_mesh,
    scratch_types=[
        pltpu.VMEM((2, CH, GSZ), jnp.int32),
        pltpu.VMEM((2, CH, GSZ), jnp.int32),
        pltpu.VMEM((2, CH, GSZ, D_MSG), jnp.float32),
        pltpu.VMEM_SHARED((NP, D_MSG), jnp.float32),
        pltpu.VMEM_SHARED((NP, D_MSG), jnp.float32),
        pltpu.SemaphoreType.DMA,
        pltpu.SemaphoreType.DMA,
    ],
    compiler_params=pltpu.CompilerParams(use_tc_tiling_on_sc=False),
)


def _deg_body(dst_hbm, ones_hbm, zeros_hbm, out_hbm, dstv, onesb, acc_sh,
              ssem):
    cid = lax.axis_index("c")
    sid = lax.axis_index("s")
    wid = sid * NC + cid
    base = wid * GPW

    pltpu.sync_copy(zeros_hbm.at[pl.ds(sid * RPW, RPW)],
                    acc_sh.at[pl.ds(sid * RPW, RPW)])
    pltpu.sync_copy(ones_hbm, onesb)
    plsc.subcore_barrier()

    s_prev = None
    for c in range(NCHUNK):
        b = c % 2
        pltpu.sync_copy(dst_hbm.at[pl.ds(base + c * CH, CH)], dstv.at[b])
        s = [pltpu.async_copy(onesb, acc_sh.at[dstv.at[b].at[j]], ssem,
                              add=True)
             for j in range(CH)]
        if s_prev is not None:
            for d_ in s_prev:
                d_.wait()
        s_prev = s
    for d_ in s_prev:
        d_.wait()

    plsc.subcore_barrier()
    pltpu.sync_copy(acc_sh.at[pl.ds(sid * RPW, RPW)],
                    out_hbm.at[cid].at[pl.ds(sid * RPW, RPW)])


_deg_kernel = pl.kernel(
    _deg_body,
    out_type=jax.ShapeDtypeStruct((NC, NP, D_MSG), jnp.float32),
    mesh=_mesh,
    scratch_types=[
        pltpu.VMEM((2, CH, GSZ), jnp.int32),
        pltpu.VMEM((GSZ, D_MSG), jnp.float32),
        pltpu.VMEM_SHARED((NP, D_MSG), jnp.float32),
        pltpu.SemaphoreType.DMA,
    ],
    compiler_params=pltpu.CompilerParams(use_tc_tiling_on_sc=False),
)


def _mm1_body(x_ref, w_ref, o_ref):
    o_ref[...] = jnp.dot(x_ref[...], w_ref[...],
                         preferred_element_type=jnp.float32)


def _prep_body(degp_ref, h1_ref, dinv_ref, hs_ref):
    deg = degp_ref[0] + degp_ref[1] + 1.0
    dinv = lax.rsqrt(jnp.maximum(deg, 1.0))
    dinv_ref[...] = dinv
    hs_ref[...] = h1_ref[...] * dinv


def _mid_body(acc_ref, hs_ref, dinv_ref, g_ref):
    t = (acc_ref[0] + acc_ref[1] + hs_ref[...]) * dinv_ref[...]
    g_ref[...] = jnp.maximum(t, 0.0) * dinv_ref[...]


def _fin_body(acc_ref, g_ref, dinv_ref, w2_ref, o_ref):
    a = (acc_ref[0] + acc_ref[1] + g_ref[...]) * dinv_ref[...]
    o_ref[...] = jnp.dot(a, w2_ref[...], preferred_element_type=jnp.float32)


def kernel(x, adj, W1, W2):
    n = x.shape[0]
    e = adj.shape[1]
    d_out = W2.shape[1]
    src = adj[0].astype(jnp.int32)
    dst = adj[1].astype(jnp.int32)
    pad = EP - e
    src2d = jnp.concatenate([src, jnp.full((pad,), n, jnp.int32)]).reshape(
        GROUPS, GSZ)
    dst2d = jnp.concatenate([dst, jnp.full((pad,), n, jnp.int32)]).reshape(
        GROUPS, GSZ)
    zeros = jnp.zeros((NP, D_MSG), jnp.float32)
    ones = jnp.ones((GSZ, D_MSG), jnp.float32)

    degp = _deg_kernel(dst2d, ones, zeros)
    h1 = pl.pallas_call(
        _mm1_body,
        out_shape=jax.ShapeDtypeStruct((n, D_MSG), jnp.float32),
    )(x, W1)
    h1p = jnp.concatenate([h1, jnp.zeros((NP - n, D_MSG), jnp.float32)])

    dinvb, hs1 = pl.pallas_call(
        _prep_body,
        out_shape=(jax.ShapeDtypeStruct((NP, D_MSG), jnp.float32),
                   jax.ShapeDtypeStruct((NP, D_MSG), jnp.float32)),
    )(degp, h1p)

    acc1 = _msg_kernel(hs1, src2d, dst2d, zeros)

    g = pl.pallas_call(
        _mid_body,
        out_shape=jax.ShapeDtypeStruct((NP, D_MSG), jnp.float32),
    )(acc1, hs1, dinvb)

    acc2 = _msg_kernel(g, src2d, dst2d, zeros)

    out = pl.pallas_call(
        _fin_body,
        out_shape=jax.ShapeDtypeStruct((NP, d_out), jnp.float32),
    )(acc2, g, dinvb, W2)
    return out[:n]


# speedup vs baseline: 52.7279x; 1.0064x over previous
import functools

import jax
import jax.numpy as jnp
from jax import lax
from jax.experimental import pallas as pl
from jax.experimental.pallas import tpu as pltpu
from jax.experimental.pallas import tpu_sc as plsc

N_NODES = 10000
N_EDGES = 320000
D_MSG = 16

NP = 10240
EP = 327680
GSZ = 128
GROUPS = EP // GSZ
NC, NS = 2, 16
NW = NC * NS
GPW = GROUPS // NW
CH = 8
NCHUNK = GPW // CH
RPW = NP // NS

_mesh = plsc.VectorSubcoreMesh(core_axis_name="c", subcore_axis_name="s")


def _msg_body(hs_hbm, src_hbm, dst_hbm, zeros_hbm, out_hbm,
              srcv, dstv, rowsbuf, acc_sh, table_sh, gsem, ssem):
    cid = lax.axis_index("c")
    sid = lax.axis_index("s")
    wid = sid * NC + cid
    base = wid * GPW

    pltpu.sync_copy(zeros_hbm.at[pl.ds(sid * RPW, RPW)],
                    acc_sh.at[pl.ds(sid * RPW, RPW)])
    pltpu.sync_copy(hs_hbm.at[pl.ds(sid * RPW, RPW)],
                    table_sh.at[pl.ds(sid * RPW, RPW)])
    pltpu.sync_copy(src_hbm.at[pl.ds(base, GPW)], srcv)
    pltpu.sync_copy(dst_hbm.at[pl.ds(base, GPW)], dstv)
    plsc.subcore_barrier()

    def fire_gathers(c, b):
        return [pltpu.async_copy(table_sh.at[srcv.at[c * CH + j]],
                                 rowsbuf.at[b].at[j], gsem)
                for j in range(CH)]

    def fire_scatters(c, b):
        return [pltpu.async_copy(rowsbuf.at[b].at[j],
                                 acc_sh.at[dstv.at[c * CH + j]], ssem,
                                 add=True)
                for j in range(CH)]

    g = fire_gathers(0, 0)
    s_prev = None
    for c in range(NCHUNK):
        b = c % 2
        for d_ in g:
            d_.wait()
        s = fire_scatters(c, b)
        if s_prev is not None:
            for d_ in s_prev:
                d_.wait()
        if c + 1 < NCHUNK:
            g = fire_gathers(c + 1, 1 - b)
        s_prev = s
    for d_ in s_prev:
        d_.wait()

    plsc.subcore_barrier()
    pltpu.sync_copy(acc_sh.at[pl.ds(sid * RPW, RPW)],
                    out_hbm.at[cid].at[pl.ds(sid * RPW, RPW)])


_msg_kernel = pl.kernel(
    _msg_body,
    out_type=jax.ShapeDtypeStruct((NC, NP, D_MSG), jnp.float32),
    mesh=_mesh,
    scratch_types=[
        pltpu.VMEM((GPW, GSZ), jnp.int32),
        pltpu.VMEM((GPW, GSZ), jnp.int32),
        pltpu.VMEM((2, CH, GSZ, D_MSG), jnp.float32),
        pltpu.VMEM_SHARED((NP, D_MSG), jnp.float32),
        pltpu.VMEM_SHARED((NP, D_MSG), jnp.float32),
        pltpu.SemaphoreType.DMA,
        pltpu.SemaphoreType.DMA,
    ],
    compiler_params=pltpu.CompilerParams(use_tc_tiling_on_sc=False),
)


def _deg_body(dst_hbm, ones_hbm, zeros_hbm, out_hbm, dstv, onesb, acc_sh,
              ssem):
    cid = lax.axis_index("c")
    sid = lax.axis_index("s")
    wid = sid * NC + cid
    base = wid * GPW

    pltpu.sync_copy(zeros_hbm.at[pl.ds(sid * RPW, RPW)],
                    acc_sh.at[pl.ds(sid * RPW, RPW)])
    pltpu.sync_copy(ones_hbm, onesb)
    pltpu.sync_copy(dst_hbm.at[pl.ds(base, GPW)], dstv)
    plsc.subcore_barrier()

    s_prev = None
    for c in range(NCHUNK):
        s = [pltpu.async_copy(onesb, acc_sh.at[dstv.at[c * CH + j]], ssem,
                              add=True)
             for j in range(CH)]
        if s_prev is not None:
            for d_ in s_prev:
                d_.wait()
        s_prev = s
    for d_ in s_prev:
        d_.wait()

    plsc.subcore_barrier()
    pltpu.sync_copy(acc_sh.at[pl.ds(sid * RPW, RPW)],
                    out_hbm.at[cid].at[pl.ds(sid * RPW, RPW)])


_deg_kernel = pl.kernel(
    _deg_body,
    out_type=jax.ShapeDtypeStruct((NC, NP, D_MSG), jnp.float32),
    mesh=_mesh,
    scratch_types=[
        pltpu.VMEM((GPW, GSZ), jnp.int32),
        pltpu.VMEM((GSZ, D_MSG), jnp.float32),
        pltpu.VMEM_SHARED((NP, D_MSG), jnp.float32),
        pltpu.SemaphoreType.DMA,
    ],
    compiler_params=pltpu.CompilerParams(use_tc_tiling_on_sc=False),
)


def _mm1_body(x_ref, w_ref, o_ref):
    o_ref[...] = jnp.dot(x_ref[...], w_ref[...],
                         preferred_element_type=jnp.float32)


def _prep_body(degp_ref, h1_ref, dinv_ref, hs_ref):
    deg = degp_ref[0] + degp_ref[1] + 1.0
    dinv = lax.rsqrt(jnp.maximum(deg, 1.0))
    dinv_ref[...] = dinv
    hs_ref[...] = h1_ref[...] * dinv


def _mid_body(acc_ref, hs_ref, dinv_ref, g_ref):
    t = (acc_ref[0] + acc_ref[1] + hs_ref[...]) * dinv_ref[...]
    g_ref[...] = jnp.maximum(t, 0.0) * dinv_ref[...]


def _fin_body(acc_ref, g_ref, dinv_ref, w2_ref, o_ref):
    a = (acc_ref[0] + acc_ref[1] + g_ref[...]) * dinv_ref[...]
    o_ref[...] = jnp.dot(a, w2_ref[...], preferred_element_type=jnp.float32)


def kernel(x, adj, W1, W2):
    n = x.shape[0]
    e = adj.shape[1]
    d_out = W2.shape[1]
    src = adj[0].astype(jnp.int32)
    dst = adj[1].astype(jnp.int32)
    pad = EP - e
    src2d = jnp.concatenate([src, jnp.full((pad,), n, jnp.int32)]).reshape(
        GROUPS, GSZ)
    dst2d = jnp.concatenate([dst, jnp.full((pad,), n, jnp.int32)]).reshape(
        GROUPS, GSZ)
    zeros = jnp.zeros((NP, D_MSG), jnp.float32)
    ones = jnp.ones((GSZ, D_MSG), jnp.float32)

    degp = _deg_kernel(dst2d, ones, zeros)
    h1 = pl.pallas_call(
        _mm1_body,
        out_shape=jax.ShapeDtypeStruct((n, D_MSG), jnp.float32),
    )(x, W1)
    h1p = jnp.concatenate([h1, jnp.zeros((NP - n, D_MSG), jnp.float32)])

    dinvb, hs1 = pl.pallas_call(
        _prep_body,
        out_shape=(jax.ShapeDtypeStruct((NP, D_MSG), jnp.float32),
                   jax.ShapeDtypeStruct((NP, D_MSG), jnp.float32)),
    )(degp, h1p)

    acc1 = _msg_kernel(hs1, src2d, dst2d, zeros)

    g = pl.pallas_call(
        _mid_body,
        out_shape=jax.ShapeDtypeStruct((NP, D_MSG), jnp.float32),
    )(acc1, hs1, dinvb)

    acc2 = _msg_kernel(g, src2d, dst2d, zeros)

    out = pl.pallas_call(
        _fin_body,
        out_shape=jax.ShapeDtypeStruct((NP, d_out), jnp.float32),
    )(acc2, g, dinvb, W2)
    return out[:n]
